# Initial kernel scaffold; baseline (speedup 1.0000x reference)
import jax
import jax.numpy as jnp
from jax.experimental import pallas as pl


def kernel(x, edge_index, batch, W1, b1, g1, be1, W2, b2, g2, be2, fW1, fb1, fW2, fb2):
    raise NotImplementedError("write your pallas kernel here")


# speedup vs baseline: 19.3804x; 19.3804x over previous
import jax
import jax.numpy as jnp
from jax import lax
from jax.experimental import pallas as pl
from jax.experimental.pallas import tpu as pltpu
from jax.experimental.pallas import tpu_sc as plsc

_N = 10000
_E = 320000
_DIN = 128
_DH = 64
_B = 64
_EPS = 1e-5

_NC = 2
_NS = 16
_NW = _NC * _NS
_CH = 128
_CPW = -(-_E // (_CH * _NW))
_EPAD = _NW * _CPW * _CH
_NPAD = 10112
_RPT = _NPAD // _NS
_DW = 16

_sc_mesh = plsc.VectorSubcoreMesh(core_axis_name="c", subcore_axis_name="s")
_sc_params = pltpu.CompilerParams(use_tc_tiling_on_sc=False)


def _sc_deg_body(dst_hbm, ones_hbm, zeros_hbm, out_hbm, dst_v, ones_v, acc_sh):
    cid = lax.axis_index("c")
    sid = lax.axis_index("s")
    pltpu.sync_copy(zeros_hbm, acc_sh.at[pl.ds(sid * _RPT, _RPT)])
    pltpu.sync_copy(ones_hbm, ones_v)
    wid = sid * _NC + cid
    pltpu.sync_copy(dst_hbm.at[wid], dst_v)
    plsc.subcore_barrier()

    def body(j, carry):
        pltpu.sync_copy(ones_v, acc_sh.at[dst_v.at[j]], add=True)
        return carry

    lax.fori_loop(0, _CPW, body, 0)
    plsc.subcore_barrier()
    pltpu.sync_copy(acc_sh.at[pl.ds(sid * _RPT, _RPT)],
                    out_hbm.at[cid, pl.ds(sid * _RPT, _RPT)])


_deg_kernel = pl.kernel(
    _sc_deg_body,
    out_type=jax.ShapeDtypeStruct((_NC, _NPAD, _DW), jnp.float32),
    mesh=_sc_mesh,
    scratch_types=[
        pltpu.VMEM((_CPW, _CH), jnp.int32),
        pltpu.VMEM((_CH, _DW), jnp.float32),
        pltpu.VMEM_SHARED((_NPAD, _DW), jnp.float32),
    ],
    compiler_params=_sc_params,
)


def _sc_spmm_body(hn_hbm, src_hbm, dst_hbm, zeros_hbm, out_hbm,
                  src_v, dst_v, rows_v, sem, acc_sh):
    cid = lax.axis_index("c")
    sid = lax.axis_index("s")
    pltpu.sync_copy(zeros_hbm, acc_sh.at[pl.ds(sid * _RPT, _RPT)])
    wid = sid * _NC + cid
    pltpu.sync_copy(src_hbm.at[wid], src_v)
    pltpu.sync_copy(dst_hbm.at[wid], dst_v)
    plsc.subcore_barrier()

    def body(j, carry):
        pltpu.async_copy(hn_hbm.at[src_v.at[j]], rows_v, sem).wait()
        pltpu.sync_copy(rows_v, acc_sh.at[dst_v.at[j]], add=True)
        return carry

    lax.fori_loop(0, _CPW, body, 0)
    plsc.subcore_barrier()
    pltpu.sync_copy(acc_sh.at[pl.ds(sid * _RPT, _RPT)],
                    out_hbm.at[cid, pl.ds(sid * _RPT, _RPT)])


_spmm_kernel = pl.kernel(
    _sc_spmm_body,
    out_type=jax.ShapeDtypeStruct((_NC, _NPAD, _DH), jnp.float32),
    mesh=_sc_mesh,
    scratch_types=[
        pltpu.VMEM((_CPW, _CH), jnp.int32),
        pltpu.VMEM((_CPW, _CH), jnp.int32),
        pltpu.VMEM((_CH, _DH), jnp.float32),
        pltpu.SemaphoreType.DMA,
        pltpu.VMEM_SHARED((_NPAD, _DH), jnp.float32),
    ],
    compiler_params=_sc_params,
)


def _tc1_body(x_ref, w1_ref, degp_ref, hn_ref, rmat_ref):
    deg = degp_ref[0, :_N, 0:1] + degp_ref[1, :_N, 0:1] + 1.0
    rmat = jnp.broadcast_to(lax.rsqrt(deg), (_N, _DH))
    h = jnp.dot(x_ref[...], w1_ref[...], preferred_element_type=jnp.float32)
    hn_ref[...] = h * rmat
    rmat_ref[...] = rmat


_tc1 = pl.pallas_call(
    _tc1_body,
    out_shape=[
        jax.ShapeDtypeStruct((_N, _DH), jnp.float32),
        jax.ShapeDtypeStruct((_N, _DH), jnp.float32),
    ],
)


def _tc2_body(p_ref, hn_ref, rmat_ref, b1_ref, g1_ref, be1_ref, w2_ref, out_ref):
    rmat = rmat_ref[...]
    a = rmat * (p_ref[0, :_N, :] + p_ref[1, :_N, :] + hn_ref[...]) + b1_ref[...]
    m = jnp.mean(a, axis=0, keepdims=True)
    c = a - m
    v = jnp.mean(c * c, axis=0, keepdims=True)
    h = jnp.maximum(c * lax.rsqrt(v + _EPS) * g1_ref[...] + be1_ref[...], 0.0)
    out_ref[...] = jnp.dot(h, w2_ref[...], preferred_element_type=jnp.float32) * rmat


_tc2 = pl.pallas_call(
    _tc2_body,
    out_shape=jax.ShapeDtypeStruct((_N, _DH), jnp.float32),
)


def _tc3_body(p_ref, hn_ref, rmat_ref, b2_ref, g2_ref, be2_ref, batch_ref,
              fw1_ref, fb1_ref, fw2_ref, fb2_ref, out_ref):
    a = rmat_ref[...] * (p_ref[0, :_N, :] + p_ref[1, :_N, :] + hn_ref[...]) + b2_ref[...]
    m = jnp.mean(a, axis=0, keepdims=True)
    c = a - m
    v = jnp.mean(c * c, axis=0, keepdims=True)
    h = jnp.maximum(c * lax.rsqrt(v + _EPS) * g2_ref[...] + be2_ref[...], 0.0)
    seg = lax.broadcasted_iota(jnp.int32, (_B, _N), 0)
    maskt = (batch_ref[...] == seg).astype(jnp.float32)
    s = jnp.dot(maskt, h, preferred_element_type=jnp.float32)
    cnt = jnp.sum(maskt, axis=1, keepdims=True)
    pool = s / jnp.maximum(cnt, 1.0)
    z = jnp.maximum(
        jnp.dot(pool, fw1_ref[...], preferred_element_type=jnp.float32) + fb1_ref[...],
        0.0)
    logits = jnp.dot(z, fw2_ref[...], preferred_element_type=jnp.float32) + fb2_ref[...]
    out_ref[...] = jax.nn.sigmoid(logits)


_tc3 = pl.pallas_call(
    _tc3_body,
    out_shape=jax.ShapeDtypeStruct((_B, 1), jnp.float32),
)


def kernel(x, edge_index, batch, W1, b1, g1, be1, W2, b2, g2, be2, fW1, fb1, fW2, fb2):
    src = edge_index[0]
    dst = edge_index[1]
    pad_e = _EPAD - _E
    srcp = jnp.concatenate([src, jnp.zeros((pad_e,), jnp.int32)]).reshape(_NW, _CPW, _CH)
    dstp = jnp.concatenate([dst, jnp.full((pad_e,), _NPAD - 1, jnp.int32)]).reshape(_NW, _CPW, _CH)
    zeros64 = jnp.zeros((_RPT, _DH), jnp.float32)
    zeros16 = jnp.zeros((_RPT, _DW), jnp.float32)
    ones16 = jnp.ones((_CH, _DW), jnp.float32)

    degp = _deg_kernel(dstp, ones16, zeros16)
    hn1, rmat = _tc1(x, W1, degp)
    p1 = _spmm_kernel(hn1, srcp, dstp, zeros64)
    hn2 = _tc2(p1, hn1, rmat, b1.reshape(1, -1), g1.reshape(1, -1),
               be1.reshape(1, -1), W2)
    p2 = _spmm_kernel(hn2, srcp, dstp, zeros64)
    out = _tc3(p2, hn2, rmat, b2.reshape(1, -1), g2.reshape(1, -1),
               be2.reshape(1, -1), batch.reshape(1, -1), fW1,
               fb1.reshape(1, -1), fW2, fb2.reshape(1, 1))
    return out
